# baseline (speedup 1.0000x reference)
import functools

import jax
import jax.numpy as jnp
from jax import lax
from jax.experimental import pallas as pl
from jax.experimental.pallas import tpu as pltpu
from jax.experimental.pallas import tpu_sc as plsc

N = 10000
E = 320000
D = 128
DE = 16
SIGMAS = [1.5 ** x for x in range(15)]
NEG_SLOPE = 0.01
SKIP_W = 0.5

C = 128
NCHUNKS = E // C
NWORK = 32
STEPS = (NCHUNKS + NWORK - 1) // NWORK
ROWS8 = 624

_R1 = 1000


def _node_pre_body(h_ref, wa_ref, wb_ref, oa_ref, ob_ref):
    h = h_ref[...]
    oa_ref[...] = jnp.dot(h, wa_ref[...], preferred_element_type=jnp.float32)
    ob_ref[...] = jnp.dot(h, wb_ref[...], preferred_element_type=jnp.float32)


def _node_pre(lig_h, wa, wb):
    return pl.pallas_call(
        _node_pre_body,
        grid=(N // _R1,),
        in_specs=[
            pl.BlockSpec((_R1, D), lambda i: (i, 0)),
            pl.BlockSpec((D, D), lambda i: (0, 0)),
            pl.BlockSpec((D, D), lambda i: (0, 0)),
        ],
        out_specs=[
            pl.BlockSpec((_R1, D), lambda i: (i, 0)),
            pl.BlockSpec((_R1, D), lambda i: (i, 0)),
        ],
        out_shape=[
            jax.ShapeDtypeStruct((N, D), jnp.float32),
            jax.ShapeDtypeStruct((N, D), jnp.float32),
        ],
    )(lig_h, wa, wb)


def _dist_body(src_hbm, dst_hbm, c4_hbm, d2_hbm, idxs, idxd, cflat, d2buf):
    cid = lax.axis_index("c")
    sid = lax.axis_index("s")
    wid = sid * 2 + cid

    pltpu.sync_copy(c4_hbm, cflat)

    def step(k, carry):
        chunk = wid + k * NWORK

        @pl.when(chunk < NCHUNKS)
        def _():
            base = chunk * C
            pltpu.sync_copy(src_hbm.at[pl.ds(base, C)], idxs)
            pltpu.sync_copy(dst_hbm.at[pl.ds(base, C)], idxd)
            for g in range(C // 16):
                s16 = idxs[pl.ds(g * 16, 16)] * 4
                t16 = idxd[pl.ds(g * 16, 16)] * 4
                dx = (plsc.load_gather(cflat, [s16])
                      - plsc.load_gather(cflat, [t16]))
                dy = (plsc.load_gather(cflat, [s16 + 1])
                      - plsc.load_gather(cflat, [t16 + 1]))
                dz = (plsc.load_gather(cflat, [s16 + 2])
                      - plsc.load_gather(cflat, [t16 + 2]))
                d2buf[pl.ds(g * 16, 16)] = dx * dx + dy * dy + dz * dz
            pltpu.sync_copy(d2buf, d2_hbm.at[pl.ds(base, C)])

        return carry

    lax.fori_loop(0, STEPS, step, 0)


def _dist(src, dst, coords4):
    mesh = plsc.VectorSubcoreMesh(core_axis_name="c", subcore_axis_name="s")
    return pl.kernel(
        _dist_body,
        out_type=jax.ShapeDtypeStruct((E,), jnp.float32),
        mesh=mesh,
        scratch_types=[
            pltpu.VMEM((C,), jnp.int32),
            pltpu.VMEM((C,), jnp.int32),
            pltpu.VMEM((4 * N,), jnp.float32),
            pltpu.VMEM((C,), jnp.float32),
        ],
        compiler_params=pltpu.CompilerParams(needs_layout_passes=False),
    )(src, dst, coords4)


def _gather_body(ta_hbm, tb_hbm, src_hbm, dst_hbm, gs_hbm, gd_hbm,
                 idxs, idxd, bufs, bufd, sems, semd):
    cid = lax.axis_index("c")
    sid = lax.axis_index("s")
    wid = sid * 2 + cid

    def step(k, carry):
        chunk = wid + k * NWORK

        @pl.when(chunk < NCHUNKS)
        def _():
            base = chunk * C
            pltpu.sync_copy(src_hbm.at[pl.ds(base, C)], idxs)
            pltpu.sync_copy(dst_hbm.at[pl.ds(base, C)], idxd)
            cp1 = pltpu.async_copy(ta_hbm.at[idxs], bufs, sems)
            cp2 = pltpu.async_copy(tb_hbm.at[idxd], bufd, semd)
            cp1.wait()
            cp2.wait()
            pltpu.sync_copy(bufs, gs_hbm.at[pl.ds(base, C)])
            pltpu.sync_copy(bufd, gd_hbm.at[pl.ds(base, C)])

        return carry

    lax.fori_loop(0, STEPS, step, 0)


def _gather(ta, tb, src, dst):
    mesh = plsc.VectorSubcoreMesh(core_axis_name="c", subcore_axis_name="s")
    return pl.kernel(
        _gather_body,
        out_type=(
            jax.ShapeDtypeStruct((E, D), jnp.float32),
            jax.ShapeDtypeStruct((E, D), jnp.float32),
        ),
        mesh=mesh,
        scratch_types=[
            pltpu.VMEM((C,), jnp.int32),
            pltpu.VMEM((C,), jnp.int32),
            pltpu.VMEM((C, D), jnp.float32),
            pltpu.VMEM((C, D), jnp.float32),
            pltpu.SemaphoreType.DMA,
            pltpu.SemaphoreType.DMA,
        ],
        compiler_params=pltpu.CompilerParams(needs_layout_passes=False),
    )(ta, tb, src, dst)


_B3 = 2560


def _extra_body(d2_ref, ea_ref, coef_ref, w1cd_ref, out_ref):
    rbf1 = jnp.exp(d2_ref[...] * coef_ref[...])
    cfeat = jnp.concatenate([ea_ref[...], rbf1], axis=1)
    out_ref[...] = jnp.dot(cfeat, w1cd_ref[...],
                           preferred_element_type=jnp.float32)


def _extra(d2x16, eattr, coef, w1cd):
    return pl.pallas_call(
        _extra_body,
        grid=(E // _B3,),
        in_specs=[
            pl.BlockSpec((_B3, DE), lambda i: (i, 0)),
            pl.BlockSpec((_B3, DE), lambda i: (i, 0)),
            pl.BlockSpec((1, DE), lambda i: (0, 0)),
            pl.BlockSpec((32, D), lambda i: (0, 0)),
        ],
        out_specs=pl.BlockSpec((_B3, D), lambda i: (i, 0)),
        out_shape=jax.ShapeDtypeStruct((E, D), jnp.float32),
    )(d2x16, eattr, coef, w1cd)


def _edge_mlp_body(gs_ref, gd_ref, ex_ref, w2_ref, gam_ref, bet_ref,
                   b2_ref, out_ref):
    pre = gs_ref[...] + gd_ref[...] + ex_ref[...]
    pre = jnp.where(pre >= 0, pre, NEG_SLOPE * pre)
    mu = jnp.mean(pre, axis=1, keepdims=True)
    xc = pre - mu
    var = jnp.mean(xc * xc, axis=1, keepdims=True)
    hdd = xc * lax.rsqrt(var + 1e-5) * gam_ref[...] + bet_ref[...]
    msg = jnp.dot(hdd, w2_ref[...], preferred_element_type=jnp.float32)
    out_ref[...] = msg + b2_ref[...]


def _edge_mlp(gs, gd, extra, w2, gamma, beta, b2):
    return pl.pallas_call(
        _edge_mlp_body,
        grid=(E // _B3,),
        in_specs=[
            pl.BlockSpec((_B3, D), lambda i: (i, 0)),
            pl.BlockSpec((_B3, D), lambda i: (i, 0)),
            pl.BlockSpec((_B3, D), lambda i: (i, 0)),
            pl.BlockSpec((D, D), lambda i: (0, 0)),
            pl.BlockSpec((1, D), lambda i: (0, 0)),
            pl.BlockSpec((1, D), lambda i: (0, 0)),
            pl.BlockSpec((1, D), lambda i: (0, 0)),
        ],
        out_specs=pl.BlockSpec((_B3, D), lambda i: (i, 0)),
        out_shape=jax.ShapeDtypeStruct((E, D), jnp.float32),
    )(gs, gd, extra, w2, gamma, beta, b2)


def _scatter_body(msg_hbm, dst_hbm, zer_hbm, out_hbm, hist_hbm,
                  idxv, bufv, hist, acc):
    cid = lax.axis_index("c")
    sid = lax.axis_index("s")
    wid = sid * 2 + cid
    row0 = sid * ROWS8

    pltpu.sync_copy(zer_hbm.at[pl.ds(row0, ROWS8)],
                    acc.at[pl.ds(row0, ROWS8)])

    @pl.when(sid == 15)
    def _():
        pltpu.sync_copy(zer_hbm.at[pl.ds(16 * ROWS8, N - 16 * ROWS8)],
                        acc.at[pl.ds(16 * ROWS8, N - 16 * ROWS8)])

    def zstep(i, carry):
        hist[pl.ds(i * 16, 16)] = jnp.zeros((16,), jnp.float32)
        return carry

    lax.fori_loop(0, N // 16, zstep, 0)
    plsc.subcore_barrier()

    ones16 = jnp.ones((16,), jnp.float32)

    def step(k, carry):
        chunk = wid + k * NWORK

        @pl.when(chunk < NCHUNKS)
        def _():
            base = chunk * C
            pltpu.sync_copy(dst_hbm.at[pl.ds(base, C)], idxv)
            pltpu.sync_copy(msg_hbm.at[pl.ds(base, C)], bufv)
            pltpu.sync_copy(bufv, acc.at[idxv], add=True)
            for g in range(C // 16):
                i16 = idxv[pl.ds(g * 16, 16)]
                plsc.addupdate_scatter(hist, [i16], ones16)

        return carry

    lax.fori_loop(0, STEPS, step, 0)
    pltpu.sync_copy(hist, hist_hbm.at[pl.ds(wid * N, N)])
    plsc.subcore_barrier()
    pltpu.sync_copy(acc.at[pl.ds(row0, ROWS8)],
                    out_hbm.at[cid, pl.ds(row0, ROWS8)])

    @pl.when(sid == 15)
    def _():
        pltpu.sync_copy(acc.at[pl.ds(16 * ROWS8, N - 16 * ROWS8)],
                        out_hbm.at[cid, pl.ds(16 * ROWS8, N - 16 * ROWS8)])


def _scatter(msg, dst, zeros_nd):
    mesh = plsc.VectorSubcoreMesh(core_axis_name="c", subcore_axis_name="s")
    return pl.kernel(
        _scatter_body,
        out_type=(
            jax.ShapeDtypeStruct((2, N, D), jnp.float32),
            jax.ShapeDtypeStruct((NWORK * N,), jnp.float32),
        ),
        mesh=mesh,
        scratch_types=[
            pltpu.VMEM((C,), jnp.int32),
            pltpu.VMEM((C, D), jnp.float32),
            pltpu.VMEM((N,), jnp.float32),
            pltpu.VMEM_SHARED((N, D), jnp.float32),
        ],
        compiler_params=pltpu.CompilerParams(needs_layout_passes=False),
    )(msg, dst, zeros_nd)


_R5 = 1000


def _final_body(acc_ref, hist_ref, h_ref, out_ref):
    sums = acc_ref[0] + acc_ref[1]
    cnt = jnp.sum(hist_ref[0], axis=0)[:, None]
    agg = sums / jnp.maximum(cnt, 1.0)
    out_ref[...] = SKIP_W * agg + (1.0 - SKIP_W) * h_ref[...]


def _final(acc, hists, lig_h):
    hists_t = hists.reshape(NWORK, N // _R5, _R5).transpose(1, 0, 2)
    return pl.pallas_call(
        _final_body,
        grid=(N // _R5,),
        in_specs=[
            pl.BlockSpec((2, _R5, D), lambda i: (0, i, 0)),
            pl.BlockSpec((1, NWORK, _R5), lambda i: (i, 0, 0)),
            pl.BlockSpec((_R5, D), lambda i: (i, 0)),
        ],
        out_specs=pl.BlockSpec((_R5, D), lambda i: (i, 0)),
        out_shape=jax.ShapeDtypeStruct((N, D), jnp.float32),
    )(acc, hists_t, lig_h)


def kernel(lig_h, lig_coords, lig_edge_index, lig_edge_attr, W1, b1,
           ln_gamma, ln_beta, W2, b2):
    src = lig_edge_index[0]
    dst = lig_edge_index[1]

    wa = W1[0:D]
    wb = W1[D:2 * D]
    w1cd = jnp.concatenate([W1[2 * D:2 * D + DE], W1[2 * D + DE:],
                            b1[None, :]], axis=0)
    coef = jnp.array([[-1.0 / s for s in SIGMAS] + [0.0]], jnp.float32)
    coords4 = jnp.pad(lig_coords, ((0, 0), (0, 1))).reshape(-1)

    d2 = _dist(src, dst, coords4)
    d2x16 = jnp.broadcast_to(d2[:, None], (E, DE))

    pa, pb = _node_pre(lig_h, wa, wb)
    gs, gd = _gather(pa, pb, src, dst)

    extra = _extra(d2x16, lig_edge_attr, coef, w1cd)

    msg = _edge_mlp(gs, gd, extra, W2,
                    ln_gamma[None, :], ln_beta[None, :], b2[None, :])

    zeros_nd = jnp.zeros((N, D), jnp.float32)
    acc, hists = _scatter(msg, dst, zeros_nd)

    return _final(acc, hists, lig_h)


# speedup vs baseline: 1.0701x; 1.0701x over previous
import functools

import jax
import jax.numpy as jnp
from jax import lax
from jax.experimental import pallas as pl
from jax.experimental.pallas import tpu as pltpu
from jax.experimental.pallas import tpu_sc as plsc

N = 10000
E = 320000
D = 128
DE = 16
SIGMAS = [1.5 ** x for x in range(15)]
NEG_SLOPE = 0.01
SKIP_W = 0.5

C = 128
NCHUNKS = E // C
NWORK = 32
STEPS = (NCHUNKS + NWORK - 1) // NWORK
ROWS8 = 624

_R1 = 1000


def _node_pre_body(h_ref, wa_ref, wb_ref, oa_ref, ob_ref):
    h = h_ref[...]
    oa_ref[...] = jnp.dot(h, wa_ref[...], preferred_element_type=jnp.float32)
    ob_ref[...] = jnp.dot(h, wb_ref[...], preferred_element_type=jnp.float32)


def _node_pre(lig_h, wa, wb):
    return pl.pallas_call(
        _node_pre_body,
        grid=(N // _R1,),
        in_specs=[
            pl.BlockSpec((_R1, D), lambda i: (i, 0)),
            pl.BlockSpec((D, D), lambda i: (0, 0)),
            pl.BlockSpec((D, D), lambda i: (0, 0)),
        ],
        out_specs=[
            pl.BlockSpec((_R1, D), lambda i: (i, 0)),
            pl.BlockSpec((_R1, D), lambda i: (i, 0)),
        ],
        out_shape=[
            jax.ShapeDtypeStruct((N, D), jnp.float32),
            jax.ShapeDtypeStruct((N, D), jnp.float32),
        ],
    )(lig_h, wa, wb)


_CB = 1000
_DSTEPS = E // (NWORK * _CB)


def _dist_body(src_hbm, dst_hbm, c4_hbm, d2_hbm, idxs, idxd, cflat, d2buf):
    cid = lax.axis_index("c")
    sid = lax.axis_index("s")
    wid = sid * 2 + cid

    pltpu.sync_copy(c4_hbm, cflat)

    def step(k, carry):
        base = (wid + k * NWORK) * _CB
        pltpu.sync_copy(src_hbm.at[pl.ds(base, _CB)], idxs)
        pltpu.sync_copy(dst_hbm.at[pl.ds(base, _CB)], idxd)

        def group(g, carry2):
            o = g * 16
            s16 = idxs[pl.ds(o, 16)] * 4
            t16 = idxd[pl.ds(o, 16)] * 4
            dx = (plsc.load_gather(cflat, [s16])
                  - plsc.load_gather(cflat, [t16]))
            dy = (plsc.load_gather(cflat, [s16 + 1])
                  - plsc.load_gather(cflat, [t16 + 1]))
            dz = (plsc.load_gather(cflat, [s16 + 2])
                  - plsc.load_gather(cflat, [t16 + 2]))
            d2buf[pl.ds(o, 16)] = dx * dx + dy * dy + dz * dz
            return carry2

        lax.fori_loop(0, _CB // 16, group, 0, unroll=5)
        pltpu.sync_copy(d2buf, d2_hbm.at[pl.ds(base, _CB)])
        return carry

    lax.fori_loop(0, _DSTEPS, step, 0)


def _dist(src, dst, coords4):
    mesh = plsc.VectorSubcoreMesh(core_axis_name="c", subcore_axis_name="s")
    return pl.kernel(
        _dist_body,
        out_type=jax.ShapeDtypeStruct((E,), jnp.float32),
        mesh=mesh,
        scratch_types=[
            pltpu.VMEM((_CB,), jnp.int32),
            pltpu.VMEM((_CB,), jnp.int32),
            pltpu.VMEM((4 * N,), jnp.float32),
            pltpu.VMEM((_CB,), jnp.float32),
        ],
        compiler_params=pltpu.CompilerParams(needs_layout_passes=False),
    )(src, dst, coords4)


def _gather_body(ta_hbm, tb_hbm, src_hbm, dst_hbm, gs_hbm, gd_hbm,
                 idxs, idxd, bufs, bufd, sems, semd):
    cid = lax.axis_index("c")
    sid = lax.axis_index("s")
    wid = sid * 2 + cid

    def step(k, carry):
        chunk = wid + k * NWORK

        @pl.when(chunk < NCHUNKS)
        def _():
            base = chunk * C
            pltpu.sync_copy(src_hbm.at[pl.ds(base, C)], idxs)
            pltpu.sync_copy(dst_hbm.at[pl.ds(base, C)], idxd)
            cp1 = pltpu.async_copy(ta_hbm.at[idxs], bufs, sems)
            cp2 = pltpu.async_copy(tb_hbm.at[idxd], bufd, semd)
            cp1.wait()
            cp2.wait()
            pltpu.sync_copy(bufs, gs_hbm.at[pl.ds(base, C)])
            pltpu.sync_copy(bufd, gd_hbm.at[pl.ds(base, C)])

        return carry

    lax.fori_loop(0, STEPS, step, 0)


def _gather(ta, tb, src, dst):
    mesh = plsc.VectorSubcoreMesh(core_axis_name="c", subcore_axis_name="s")
    return pl.kernel(
        _gather_body,
        out_type=(
            jax.ShapeDtypeStruct((E, D), jnp.float32),
            jax.ShapeDtypeStruct((E, D), jnp.float32),
        ),
        mesh=mesh,
        scratch_types=[
            pltpu.VMEM((C,), jnp.int32),
            pltpu.VMEM((C,), jnp.int32),
            pltpu.VMEM((C, D), jnp.float32),
            pltpu.VMEM((C, D), jnp.float32),
            pltpu.SemaphoreType.DMA,
            pltpu.SemaphoreType.DMA,
        ],
        compiler_params=pltpu.CompilerParams(needs_layout_passes=False),
    )(ta, tb, src, dst)


_B3 = 2560


def _extra_body(d2_ref, ea_ref, coef_ref, w1cd_ref, out_ref):
    rbf1 = jnp.exp(d2_ref[...] * coef_ref[...])
    cfeat = jnp.concatenate([ea_ref[...], rbf1], axis=1)
    out_ref[...] = jnp.dot(cfeat, w1cd_ref[...],
                           preferred_element_type=jnp.float32).astype(jnp.bfloat16)


def _extra(d2x16, eattr, coef, w1cd):
    return pl.pallas_call(
        _extra_body,
        grid=(E // _B3,),
        in_specs=[
            pl.BlockSpec((_B3, DE), lambda i: (i, 0)),
            pl.BlockSpec((_B3, DE), lambda i: (i, 0)),
            pl.BlockSpec((1, DE), lambda i: (0, 0)),
            pl.BlockSpec((32, D), lambda i: (0, 0)),
        ],
        out_specs=pl.BlockSpec((_B3, D), lambda i: (i, 0)),
        out_shape=jax.ShapeDtypeStruct((E, D), jnp.bfloat16),
    )(d2x16, eattr, coef, w1cd)


def _edge_mlp_body(gs_ref, gd_ref, ex_ref, w2_ref, gam_ref, bet_ref,
                   b2_ref, out_ref):
    pre = gs_ref[...] + gd_ref[...] + ex_ref[...].astype(jnp.float32)
    pre = jnp.where(pre >= 0, pre, NEG_SLOPE * pre)
    mu = jnp.mean(pre, axis=1, keepdims=True)
    xc = pre - mu
    var = jnp.mean(xc * xc, axis=1, keepdims=True)
    hdd = xc * lax.rsqrt(var + 1e-5) * gam_ref[...] + bet_ref[...]
    msg = jnp.dot(hdd, w2_ref[...], preferred_element_type=jnp.float32)
    out_ref[...] = msg + b2_ref[...]


def _edge_mlp(gs, gd, extra, w2, gamma, beta, b2):
    return pl.pallas_call(
        _edge_mlp_body,
        grid=(E // _B3,),
        in_specs=[
            pl.BlockSpec((_B3, D), lambda i: (i, 0)),
            pl.BlockSpec((_B3, D), lambda i: (i, 0)),
            pl.BlockSpec((_B3, D), lambda i: (i, 0)),
            pl.BlockSpec((D, D), lambda i: (0, 0)),
            pl.BlockSpec((1, D), lambda i: (0, 0)),
            pl.BlockSpec((1, D), lambda i: (0, 0)),
            pl.BlockSpec((1, D), lambda i: (0, 0)),
        ],
        out_specs=pl.BlockSpec((_B3, D), lambda i: (i, 0)),
        out_shape=jax.ShapeDtypeStruct((E, D), jnp.float32),
    )(gs, gd, extra, w2, gamma, beta, b2)


def _scatter_body(msg_hbm, dst_hbm, zer_hbm, out_hbm, hist_hbm,
                  idxv, bufv, hist, acc):
    cid = lax.axis_index("c")
    sid = lax.axis_index("s")
    wid = sid * 2 + cid
    row0 = sid * ROWS8

    pltpu.sync_copy(zer_hbm.at[pl.ds(row0, ROWS8)],
                    acc.at[pl.ds(row0, ROWS8)])

    @pl.when(sid == 15)
    def _():
        pltpu.sync_copy(zer_hbm.at[pl.ds(16 * ROWS8, N - 16 * ROWS8)],
                        acc.at[pl.ds(16 * ROWS8, N - 16 * ROWS8)])

    def zstep(i, carry):
        hist[pl.ds(i * 16, 16)] = jnp.zeros((16,), jnp.float32)
        return carry

    lax.fori_loop(0, N // 16, zstep, 0)
    plsc.subcore_barrier()

    ones16 = jnp.ones((16,), jnp.float32)

    def step(k, carry):
        chunk = wid + k * NWORK

        @pl.when(chunk < NCHUNKS)
        def _():
            base = chunk * C
            pltpu.sync_copy(dst_hbm.at[pl.ds(base, C)], idxv)
            pltpu.sync_copy(msg_hbm.at[pl.ds(base, C)], bufv)
            pltpu.sync_copy(bufv, acc.at[idxv], add=True)
            for g in range(C // 16):
                i16 = idxv[pl.ds(g * 16, 16)]
                plsc.addupdate_scatter(hist, [i16], ones16)

        return carry

    lax.fori_loop(0, STEPS, step, 0)
    pltpu.sync_copy(hist, hist_hbm.at[pl.ds(wid * N, N)])
    plsc.subcore_barrier()
    pltpu.sync_copy(acc.at[pl.ds(row0, ROWS8)],
                    out_hbm.at[cid, pl.ds(row0, ROWS8)])

    @pl.when(sid == 15)
    def _():
        pltpu.sync_copy(acc.at[pl.ds(16 * ROWS8, N - 16 * ROWS8)],
                        out_hbm.at[cid, pl.ds(16 * ROWS8, N - 16 * ROWS8)])


def _scatter(msg, dst, zeros_nd):
    mesh = plsc.VectorSubcoreMesh(core_axis_name="c", subcore_axis_name="s")
    return pl.kernel(
        _scatter_body,
        out_type=(
            jax.ShapeDtypeStruct((2, N, D), jnp.float32),
            jax.ShapeDtypeStruct((NWORK * N,), jnp.float32),
        ),
        mesh=mesh,
        scratch_types=[
            pltpu.VMEM((C,), jnp.int32),
            pltpu.VMEM((C, D), jnp.float32),
            pltpu.VMEM((N,), jnp.float32),
            pltpu.VMEM_SHARED((N, D), jnp.float32),
        ],
        compiler_params=pltpu.CompilerParams(needs_layout_passes=False),
    )(msg, dst, zeros_nd)


_R5 = 1000


def _final_body(acc_ref, hist_ref, h_ref, out_ref):
    sums = acc_ref[0] + acc_ref[1]
    cnt = jnp.sum(hist_ref[0], axis=0)[:, None]
    agg = sums / jnp.maximum(cnt, 1.0)
    out_ref[...] = SKIP_W * agg + (1.0 - SKIP_W) * h_ref[...]


def _final(acc, hists, lig_h):
    hists_t = hists.reshape(NWORK, N // _R5, _R5).transpose(1, 0, 2)
    return pl.pallas_call(
        _final_body,
        grid=(N // _R5,),
        in_specs=[
            pl.BlockSpec((2, _R5, D), lambda i: (0, i, 0)),
            pl.BlockSpec((1, NWORK, _R5), lambda i: (i, 0, 0)),
            pl.BlockSpec((_R5, D), lambda i: (i, 0)),
        ],
        out_specs=pl.BlockSpec((_R5, D), lambda i: (i, 0)),
        out_shape=jax.ShapeDtypeStruct((N, D), jnp.float32),
    )(acc, hists_t, lig_h)


def kernel(lig_h, lig_coords, lig_edge_index, lig_edge_attr, W1, b1,
           ln_gamma, ln_beta, W2, b2):
    src = lig_edge_index[0]
    dst = lig_edge_index[1]

    wa = W1[0:D]
    wb = W1[D:2 * D]
    w1cd = jnp.concatenate([W1[2 * D:2 * D + DE], W1[2 * D + DE:],
                            b1[None, :]], axis=0)
    coef = jnp.array([[-1.0 / s for s in SIGMAS] + [0.0]], jnp.float32)
    coords4 = jnp.pad(lig_coords, ((0, 0), (0, 1))).reshape(-1)

    d2 = _dist(src, dst, coords4)
    d2x16 = jnp.broadcast_to(d2[:, None], (E, DE))

    pa, pb = _node_pre(lig_h, wa, wb)
    gs, gd = _gather(pa, pb, src, dst)

    extra = _extra(d2x16, lig_edge_attr, coef, w1cd)

    msg = _edge_mlp(gs, gd, extra, W2,
                    ln_gamma[None, :], ln_beta[None, :], b2[None, :])

    zeros_nd = jnp.zeros((N, D), jnp.float32)
    acc, hists = _scatter(msg, dst, zeros_nd)

    return _final(acc, hists, lig_h)


# speedup vs baseline: 1.1881x; 1.1102x over previous
import functools

import jax
import jax.numpy as jnp
from jax import lax
from jax.experimental import pallas as pl
from jax.experimental.pallas import tpu as pltpu
from jax.experimental.pallas import tpu_sc as plsc

N = 10000
E = 320000
D = 128
DE = 16
SIGMAS = [1.5 ** x for x in range(15)]
NEG_SLOPE = 0.01
SKIP_W = 0.5

C = 128
NCHUNKS = E // C
NWORK = 32
STEPS = (NCHUNKS + NWORK - 1) // NWORK
ROWS8 = 624

_R1 = 1000


def _node_pre_body(h_ref, wa_ref, wb_ref, oa_ref, ob_ref):
    h = h_ref[...]
    oa_ref[...] = jnp.dot(h, wa_ref[...], preferred_element_type=jnp.float32)
    ob_ref[...] = jnp.dot(h, wb_ref[...], preferred_element_type=jnp.float32)


def _node_pre(lig_h, wa, wb):
    return pl.pallas_call(
        _node_pre_body,
        grid=(N // _R1,),
        in_specs=[
            pl.BlockSpec((_R1, D), lambda i: (i, 0)),
            pl.BlockSpec((D, D), lambda i: (0, 0)),
            pl.BlockSpec((D, D), lambda i: (0, 0)),
        ],
        out_specs=[
            pl.BlockSpec((_R1, D), lambda i: (i, 0)),
            pl.BlockSpec((_R1, D), lambda i: (i, 0)),
        ],
        out_shape=[
            jax.ShapeDtypeStruct((N, D), jnp.float32),
            jax.ShapeDtypeStruct((N, D), jnp.float32),
        ],
    )(lig_h, wa, wb)


_CB = 1000
_DSTEPS = E // (NWORK * _CB)


def _dist_body(src_hbm, dst_hbm, c4_hbm, d2_hbm, idxs, idxd, cflat, d2buf):
    cid = lax.axis_index("c")
    sid = lax.axis_index("s")
    wid = sid * 2 + cid

    pltpu.sync_copy(c4_hbm, cflat)

    def step(k, carry):
        base = (wid + k * NWORK) * _CB
        pltpu.sync_copy(src_hbm.at[pl.ds(base, _CB)], idxs)
        pltpu.sync_copy(dst_hbm.at[pl.ds(base, _CB)], idxd)

        def group(g, carry2):
            o = g * 16
            s16 = idxs[pl.ds(o, 16)] * 4
            t16 = idxd[pl.ds(o, 16)] * 4
            dx = (plsc.load_gather(cflat, [s16])
                  - plsc.load_gather(cflat, [t16]))
            dy = (plsc.load_gather(cflat, [s16 + 1])
                  - plsc.load_gather(cflat, [t16 + 1]))
            dz = (plsc.load_gather(cflat, [s16 + 2])
                  - plsc.load_gather(cflat, [t16 + 2]))
            d2buf[pl.ds(o, 16)] = dx * dx + dy * dy + dz * dz
            return carry2

        lax.fori_loop(0, _CB // 16, group, 0, unroll=5)
        pltpu.sync_copy(d2buf, d2_hbm.at[pl.ds(base, _CB)])
        return carry

    lax.fori_loop(0, _DSTEPS, step, 0)


def _dist(src, dst, coords4):
    mesh = plsc.VectorSubcoreMesh(core_axis_name="c", subcore_axis_name="s")
    return pl.kernel(
        _dist_body,
        out_type=jax.ShapeDtypeStruct((E,), jnp.float32),
        mesh=mesh,
        scratch_types=[
            pltpu.VMEM((_CB,), jnp.int32),
            pltpu.VMEM((_CB,), jnp.int32),
            pltpu.VMEM((4 * N,), jnp.float32),
            pltpu.VMEM((_CB,), jnp.float32),
        ],
        compiler_params=pltpu.CompilerParams(needs_layout_passes=False),
    )(src, dst, coords4)


def _gather_body(chunk0, nchunks, ta_hbm, tb_hbm, src_hbm, dst_hbm,
                 gs_hbm, gd_hbm, idxs, idxd, bufs, bufd, sems, semd):
    cid = lax.axis_index("c")
    sid = lax.axis_index("s")
    wid = sid * 2 + cid

    def step(k, carry):
        chunk = wid + k * NWORK

        @pl.when(chunk < nchunks)
        def _():
            gbase = (chunk0 + chunk) * C
            base = chunk * C
            pltpu.sync_copy(src_hbm.at[pl.ds(gbase, C)], idxs)
            pltpu.sync_copy(dst_hbm.at[pl.ds(gbase, C)], idxd)
            cp1 = pltpu.async_copy(ta_hbm.at[idxs], bufs, sems)
            cp2 = pltpu.async_copy(tb_hbm.at[idxd], bufd, semd)
            cp1.wait()
            cp2.wait()
            pltpu.sync_copy(bufs, gs_hbm.at[pl.ds(base, C)])
            pltpu.sync_copy(bufd, gd_hbm.at[pl.ds(base, C)])

        return carry

    lax.fori_loop(0, (nchunks + NWORK - 1) // NWORK, step, 0)


def _gather(ta, tb, src, dst, e0, ne):
    mesh = plsc.VectorSubcoreMesh(core_axis_name="c", subcore_axis_name="s")
    return pl.kernel(
        functools.partial(_gather_body, e0 // C, ne // C),
        out_type=(
            jax.ShapeDtypeStruct((ne, D), jnp.float32),
            jax.ShapeDtypeStruct((ne, D), jnp.float32),
        ),
        mesh=mesh,
        scratch_types=[
            pltpu.VMEM((C,), jnp.int32),
            pltpu.VMEM((C,), jnp.int32),
            pltpu.VMEM((C, D), jnp.float32),
            pltpu.VMEM((C, D), jnp.float32),
            pltpu.SemaphoreType.DMA,
            pltpu.SemaphoreType.DMA,
        ],
        compiler_params=pltpu.CompilerParams(needs_layout_passes=False),
    )(ta, tb, src, dst)


_B3 = 3200


def _extra_body(d2_ref, ea_ref, coef_ref, w1cd_ref, out_ref):
    rbf1 = jnp.exp(d2_ref[...] * coef_ref[...])
    cfeat = jnp.concatenate([ea_ref[...], rbf1], axis=1)
    out_ref[...] = jnp.dot(cfeat, w1cd_ref[...],
                           preferred_element_type=jnp.float32).astype(jnp.bfloat16)


def _extra(d2x16, eattr, coef, w1cd, e0, ne):
    blk0 = e0 // _B3
    return pl.pallas_call(
        _extra_body,
        grid=(ne // _B3,),
        in_specs=[
            pl.BlockSpec((_B3, DE), lambda i: (blk0 + i, 0)),
            pl.BlockSpec((_B3, DE), lambda i: (blk0 + i, 0)),
            pl.BlockSpec((1, DE), lambda i: (0, 0)),
            pl.BlockSpec((32, D), lambda i: (0, 0)),
        ],
        out_specs=pl.BlockSpec((_B3, D), lambda i: (i, 0)),
        out_shape=jax.ShapeDtypeStruct((ne, D), jnp.bfloat16),
    )(d2x16, eattr, coef, w1cd)


def _edge_mlp_body(gs_ref, gd_ref, ex_ref, w2_ref, gam_ref, bet_ref,
                   b2_ref, out_ref):
    pre = gs_ref[...] + gd_ref[...] + ex_ref[...].astype(jnp.float32)
    pre = jnp.where(pre >= 0, pre, NEG_SLOPE * pre)
    mu = jnp.mean(pre, axis=1, keepdims=True)
    xc = pre - mu
    var = jnp.mean(xc * xc, axis=1, keepdims=True)
    hdd = xc * lax.rsqrt(var + 1e-5) * gam_ref[...] + bet_ref[...]
    msg = jnp.dot(hdd, w2_ref[...], preferred_element_type=jnp.float32)
    out_ref[...] = msg + b2_ref[...]


def _edge_mlp(gs, gd, extra, w2, gamma, beta, b2):
    ne = gs.shape[0]
    return pl.pallas_call(
        _edge_mlp_body,
        grid=(ne // _B3,),
        in_specs=[
            pl.BlockSpec((_B3, D), lambda i: (i, 0)),
            pl.BlockSpec((_B3, D), lambda i: (i, 0)),
            pl.BlockSpec((_B3, D), lambda i: (i, 0)),
            pl.BlockSpec((D, D), lambda i: (0, 0)),
            pl.BlockSpec((1, D), lambda i: (0, 0)),
            pl.BlockSpec((1, D), lambda i: (0, 0)),
            pl.BlockSpec((1, D), lambda i: (0, 0)),
        ],
        out_specs=pl.BlockSpec((_B3, D), lambda i: (i, 0)),
        out_shape=jax.ShapeDtypeStruct((ne, D), jnp.float32),
    )(gs, gd, extra, w2, gamma, beta, b2)


def _scatter_body(chunk0, nchunks, msg_hbm, dst_hbm, zer_hbm, out_hbm,
                  hist_hbm, idxv, bufv, hist, acc):
    cid = lax.axis_index("c")
    sid = lax.axis_index("s")
    wid = sid * 2 + cid
    row0 = sid * ROWS8

    pltpu.sync_copy(zer_hbm.at[pl.ds(row0, ROWS8)],
                    acc.at[pl.ds(row0, ROWS8)])

    @pl.when(sid == 15)
    def _():
        pltpu.sync_copy(zer_hbm.at[pl.ds(16 * ROWS8, N - 16 * ROWS8)],
                        acc.at[pl.ds(16 * ROWS8, N - 16 * ROWS8)])

    def zstep(i, carry):
        hist[pl.ds(i * 16, 16)] = jnp.zeros((16,), jnp.float32)
        return carry

    lax.fori_loop(0, N // 16, zstep, 0)
    plsc.subcore_barrier()

    ones16 = jnp.ones((16,), jnp.float32)

    def step(k, carry):
        chunk = wid + k * NWORK

        @pl.when(chunk < nchunks)
        def _():
            base = chunk * C
            pltpu.sync_copy(dst_hbm.at[pl.ds((chunk0 + chunk) * C, C)], idxv)
            pltpu.sync_copy(msg_hbm.at[pl.ds(base, C)], bufv)
            pltpu.sync_copy(bufv, acc.at[idxv], add=True)
            for g in range(C // 16):
                i16 = idxv[pl.ds(g * 16, 16)]
                plsc.addupdate_scatter(hist, [i16], ones16)

        return carry

    lax.fori_loop(0, (nchunks + NWORK - 1) // NWORK, step, 0)
    pltpu.sync_copy(hist, hist_hbm.at[pl.ds(wid * N, N)])
    plsc.subcore_barrier()
    pltpu.sync_copy(acc.at[pl.ds(row0, ROWS8)],
                    out_hbm.at[cid, pl.ds(row0, ROWS8)])

    @pl.when(sid == 15)
    def _():
        pltpu.sync_copy(acc.at[pl.ds(16 * ROWS8, N - 16 * ROWS8)],
                        out_hbm.at[cid, pl.ds(16 * ROWS8, N - 16 * ROWS8)])


def _scatter(msg, dst, zeros_nd, e0):
    ne = msg.shape[0]
    mesh = plsc.VectorSubcoreMesh(core_axis_name="c", subcore_axis_name="s")
    return pl.kernel(
        functools.partial(_scatter_body, e0 // C, ne // C),
        out_type=(
            jax.ShapeDtypeStruct((2, N, D), jnp.float32),
            jax.ShapeDtypeStruct((NWORK * N,), jnp.float32),
        ),
        mesh=mesh,
        scratch_types=[
            pltpu.VMEM((C,), jnp.int32),
            pltpu.VMEM((C, D), jnp.float32),
            pltpu.VMEM((N,), jnp.float32),
            pltpu.VMEM_SHARED((N, D), jnp.float32),
        ],
        compiler_params=pltpu.CompilerParams(needs_layout_passes=False),
    )(msg, dst, zeros_nd)


_R5 = 1000


def _final_body(acca_ref, accb_ref, hista_ref, histb_ref, h_ref, out_ref):
    sums = acca_ref[0] + acca_ref[1] + accb_ref[0] + accb_ref[1]
    cnt = (jnp.sum(hista_ref[0], axis=0)
           + jnp.sum(histb_ref[0], axis=0))[:, None]
    agg = sums / jnp.maximum(cnt, 1.0)
    out_ref[...] = SKIP_W * agg + (1.0 - SKIP_W) * h_ref[...]


def _final(acc_a, acc_b, hists_a, hists_b, lig_h):
    ha = hists_a.reshape(NWORK, N // _R5, _R5).transpose(1, 0, 2)
    hb = hists_b.reshape(NWORK, N // _R5, _R5).transpose(1, 0, 2)
    return pl.pallas_call(
        _final_body,
        grid=(N // _R5,),
        in_specs=[
            pl.BlockSpec((2, _R5, D), lambda i: (0, i, 0)),
            pl.BlockSpec((2, _R5, D), lambda i: (0, i, 0)),
            pl.BlockSpec((1, NWORK, _R5), lambda i: (i, 0, 0)),
            pl.BlockSpec((1, NWORK, _R5), lambda i: (i, 0, 0)),
            pl.BlockSpec((_R5, D), lambda i: (i, 0)),
        ],
        out_specs=pl.BlockSpec((_R5, D), lambda i: (i, 0)),
        out_shape=jax.ShapeDtypeStruct((N, D), jnp.float32),
    )(acc_a, acc_b, ha, hb, lig_h)


def kernel(lig_h, lig_coords, lig_edge_index, lig_edge_attr, W1, b1,
           ln_gamma, ln_beta, W2, b2):
    src = lig_edge_index[0]
    dst = lig_edge_index[1]

    wa = W1[0:D]
    wb = W1[D:2 * D]
    w1cd = jnp.concatenate([W1[2 * D:2 * D + DE], W1[2 * D + DE:],
                            b1[None, :]], axis=0)
    coef = jnp.array([[-1.0 / s for s in SIGMAS] + [0.0]], jnp.float32)
    coords4 = jnp.pad(lig_coords, ((0, 0), (0, 1))).reshape(-1)

    d2 = _dist(src, dst, coords4)
    d2x16 = jnp.broadcast_to(d2[:, None], (E, DE))

    pa, pb = _node_pre(lig_h, wa, wb)
    zeros_nd = jnp.zeros((N, D), jnp.float32)

    EH = E // 2
    gam, bet, b2r = ln_gamma[None, :], ln_beta[None, :], b2[None, :]

    gs1, gd1 = _gather(pa, pb, src, dst, 0, EH)
    ex1 = _extra(d2x16, lig_edge_attr, coef, w1cd, 0, EH)
    gs2, gd2 = _gather(pa, pb, src, dst, EH, EH)
    ex2 = _extra(d2x16, lig_edge_attr, coef, w1cd, EH, EH)

    msg1 = _edge_mlp(gs1, gd1, ex1, W2, gam, bet, b2r)
    acc1, hists1 = _scatter(msg1, dst, zeros_nd, 0)
    msg2 = _edge_mlp(gs2, gd2, ex2, W2, gam, bet, b2r)
    acc2, hists2 = _scatter(msg2, dst, zeros_nd, EH)

    return _final(acc1, acc2, hists1, hists2, lig_h)


# speedup vs baseline: 1.1885x; 1.0003x over previous
import functools

import jax
import jax.numpy as jnp
from jax import lax
from jax.experimental import pallas as pl
from jax.experimental.pallas import tpu as pltpu
from jax.experimental.pallas import tpu_sc as plsc

N = 10000
E = 320000
D = 128
DE = 16
SIGMAS = [1.5 ** x for x in range(15)]
NEG_SLOPE = 0.01
SKIP_W = 0.5

C = 128
NCHUNKS = E // C
NWORK = 32
STEPS = (NCHUNKS + NWORK - 1) // NWORK
ROWS8 = 624

_R1 = 1000


def _node_pre_body(h_ref, wa_ref, wb_ref, oa_ref, ob_ref):
    h = h_ref[...]
    oa_ref[...] = jnp.dot(h, wa_ref[...], preferred_element_type=jnp.float32)
    ob_ref[...] = jnp.dot(h, wb_ref[...], preferred_element_type=jnp.float32)


def _node_pre(lig_h, wa, wb):
    return pl.pallas_call(
        _node_pre_body,
        grid=(N // _R1,),
        in_specs=[
            pl.BlockSpec((_R1, D), lambda i: (i, 0)),
            pl.BlockSpec((D, D), lambda i: (0, 0)),
            pl.BlockSpec((D, D), lambda i: (0, 0)),
        ],
        out_specs=[
            pl.BlockSpec((_R1, D), lambda i: (i, 0)),
            pl.BlockSpec((_R1, D), lambda i: (i, 0)),
        ],
        out_shape=[
            jax.ShapeDtypeStruct((N, D), jnp.float32),
            jax.ShapeDtypeStruct((N, D), jnp.float32),
        ],
    )(lig_h, wa, wb)


_CB = 1000
_DSTEPS = E // (NWORK * _CB)


def _dist_body(src_hbm, dst_hbm, c4_hbm, d2_hbm, idxs, idxd, cflat, d2buf):
    cid = lax.axis_index("c")
    sid = lax.axis_index("s")
    wid = sid * 2 + cid

    pltpu.sync_copy(c4_hbm, cflat)

    def step(k, carry):
        base = (wid + k * NWORK) * _CB
        pltpu.sync_copy(src_hbm.at[pl.ds(base, _CB)], idxs)
        pltpu.sync_copy(dst_hbm.at[pl.ds(base, _CB)], idxd)

        def group(g, carry2):
            o = g * 16
            s16 = idxs[pl.ds(o, 16)] * 4
            t16 = idxd[pl.ds(o, 16)] * 4
            dx = (plsc.load_gather(cflat, [s16])
                  - plsc.load_gather(cflat, [t16]))
            dy = (plsc.load_gather(cflat, [s16 + 1])
                  - plsc.load_gather(cflat, [t16 + 1]))
            dz = (plsc.load_gather(cflat, [s16 + 2])
                  - plsc.load_gather(cflat, [t16 + 2]))
            d2buf[pl.ds(o, 16)] = dx * dx + dy * dy + dz * dz
            return carry2

        lax.fori_loop(0, _CB // 16, group, 0, unroll=5)
        pltpu.sync_copy(d2buf, d2_hbm.at[pl.ds(base, _CB)])
        return carry

    lax.fori_loop(0, _DSTEPS, step, 0)


def _dist(src, dst, coords4):
    mesh = plsc.VectorSubcoreMesh(core_axis_name="c", subcore_axis_name="s")
    return pl.kernel(
        _dist_body,
        out_type=jax.ShapeDtypeStruct((E,), jnp.float32),
        mesh=mesh,
        scratch_types=[
            pltpu.VMEM((_CB,), jnp.int32),
            pltpu.VMEM((_CB,), jnp.int32),
            pltpu.VMEM((4 * N,), jnp.float32),
            pltpu.VMEM((_CB,), jnp.float32),
        ],
        compiler_params=pltpu.CompilerParams(needs_layout_passes=False),
    )(src, dst, coords4)


def _gather_body(chunk0, nchunks, ta_hbm, tb_hbm, src_hbm, dst_hbm,
                 gs_hbm, gd_hbm, idxs, idxd, bufs, bufd, sems, semd):
    cid = lax.axis_index("c")
    sid = lax.axis_index("s")
    wid = sid * 2 + cid

    def step(k, carry):
        chunk = wid + k * NWORK

        @pl.when(chunk < nchunks)
        def _():
            gbase = (chunk0 + chunk) * C
            base = chunk * C
            pltpu.sync_copy(src_hbm.at[pl.ds(gbase, C)], idxs)
            pltpu.sync_copy(dst_hbm.at[pl.ds(gbase, C)], idxd)
            cp1 = pltpu.async_copy(ta_hbm.at[idxs], bufs, sems)
            cp2 = pltpu.async_copy(tb_hbm.at[idxd], bufd, semd)
            cp1.wait()
            cp2.wait()
            pltpu.sync_copy(bufs, gs_hbm.at[pl.ds(base, C)])
            pltpu.sync_copy(bufd, gd_hbm.at[pl.ds(base, C)])

        return carry

    lax.fori_loop(0, (nchunks + NWORK - 1) // NWORK, step, 0)


def _gather(ta, tb, src, dst, e0, ne):
    mesh = plsc.VectorSubcoreMesh(core_axis_name="c", subcore_axis_name="s")
    return pl.kernel(
        functools.partial(_gather_body, e0 // C, ne // C),
        out_type=(
            jax.ShapeDtypeStruct((ne, D), jnp.float32),
            jax.ShapeDtypeStruct((ne, D), jnp.float32),
        ),
        mesh=mesh,
        scratch_types=[
            pltpu.VMEM((C,), jnp.int32),
            pltpu.VMEM((C,), jnp.int32),
            pltpu.VMEM((C, D), jnp.float32),
            pltpu.VMEM((C, D), jnp.float32),
            pltpu.SemaphoreType.DMA,
            pltpu.SemaphoreType.DMA,
        ],
        compiler_params=pltpu.CompilerParams(needs_layout_passes=False),
    )(ta, tb, src, dst)


_B3 = 3200


def _extra_body(d2_ref, ea_ref, coef_ref, w1cd_ref, out_ref):
    rbf1 = jnp.exp(d2_ref[...] * coef_ref[...])
    cfeat = jnp.concatenate([ea_ref[...], rbf1], axis=1)
    out_ref[...] = jnp.dot(cfeat, w1cd_ref[...],
                           preferred_element_type=jnp.float32).astype(jnp.bfloat16)


def _extra(d2x16, eattr, coef, w1cd, e0, ne):
    blk0 = e0 // _B3
    return pl.pallas_call(
        _extra_body,
        grid=(ne // _B3,),
        in_specs=[
            pl.BlockSpec((_B3, DE), lambda i: (blk0 + i, 0)),
            pl.BlockSpec((_B3, DE), lambda i: (blk0 + i, 0)),
            pl.BlockSpec((1, DE), lambda i: (0, 0)),
            pl.BlockSpec((32, D), lambda i: (0, 0)),
        ],
        out_specs=pl.BlockSpec((_B3, D), lambda i: (i, 0)),
        out_shape=jax.ShapeDtypeStruct((ne, D), jnp.bfloat16),
    )(d2x16, eattr, coef, w1cd)


def _edge_mlp_body(gs_ref, gd_ref, ex_ref, w2_ref, gam_ref, bet_ref,
                   b2_ref, out_ref):
    pre = gs_ref[...] + gd_ref[...] + ex_ref[...].astype(jnp.float32)
    pre = jnp.where(pre >= 0, pre, NEG_SLOPE * pre)
    mu = jnp.mean(pre, axis=1, keepdims=True)
    xc = pre - mu
    var = jnp.mean(xc * xc, axis=1, keepdims=True)
    hdd = xc * lax.rsqrt(var + 1e-5) * gam_ref[...] + bet_ref[...]
    msg = jnp.dot(hdd, w2_ref[...], preferred_element_type=jnp.float32)
    out_ref[...] = msg + b2_ref[...]


def _edge_mlp(gs, gd, extra, w2, gamma, beta, b2):
    ne = gs.shape[0]
    return pl.pallas_call(
        _edge_mlp_body,
        grid=(ne // _B3,),
        in_specs=[
            pl.BlockSpec((_B3, D), lambda i: (i, 0)),
            pl.BlockSpec((_B3, D), lambda i: (i, 0)),
            pl.BlockSpec((_B3, D), lambda i: (i, 0)),
            pl.BlockSpec((D, D), lambda i: (0, 0)),
            pl.BlockSpec((1, D), lambda i: (0, 0)),
            pl.BlockSpec((1, D), lambda i: (0, 0)),
            pl.BlockSpec((1, D), lambda i: (0, 0)),
        ],
        out_specs=pl.BlockSpec((_B3, D), lambda i: (i, 0)),
        out_shape=jax.ShapeDtypeStruct((ne, D), jnp.float32),
    )(gs, gd, extra, w2, gamma, beta, b2)


def _scatter_body(chunk0, nchunks, msg_hbm, dst_hbm, zer_hbm, out_hbm,
                  hist_hbm, idxv, bufv, hist, acc):
    cid = lax.axis_index("c")
    sid = lax.axis_index("s")
    wid = sid * 2 + cid
    row0 = sid * ROWS8

    pltpu.sync_copy(zer_hbm.at[pl.ds(row0, ROWS8)],
                    acc.at[pl.ds(row0, ROWS8)])

    @pl.when(sid == 15)
    def _():
        pltpu.sync_copy(zer_hbm.at[pl.ds(16 * ROWS8, N - 16 * ROWS8)],
                        acc.at[pl.ds(16 * ROWS8, N - 16 * ROWS8)])

    def zstep(i, carry):
        hist[pl.ds(i * 16, 16)] = jnp.zeros((16,), jnp.float32)
        return carry

    lax.fori_loop(0, N // 16, zstep, 0)
    plsc.subcore_barrier()

    ones16 = jnp.ones((16,), jnp.float32)

    def step(k, carry):
        chunk = wid + k * NWORK

        @pl.when(chunk < nchunks)
        def _():
            base = chunk * C
            pltpu.sync_copy(dst_hbm.at[pl.ds((chunk0 + chunk) * C, C)], idxv)
            pltpu.sync_copy(msg_hbm.at[pl.ds(base, C)], bufv)
            pltpu.sync_copy(bufv, acc.at[idxv], add=True)
            for g in range(C // 16):
                i16 = idxv[pl.ds(g * 16, 16)]
                plsc.addupdate_scatter(hist, [i16], ones16)

        return carry

    lax.fori_loop(0, (nchunks + NWORK - 1) // NWORK, step, 0)
    pltpu.sync_copy(hist, hist_hbm.at[pl.ds(wid * N, N)])
    plsc.subcore_barrier()
    pltpu.sync_copy(acc.at[pl.ds(row0, ROWS8)],
                    out_hbm.at[cid, pl.ds(row0, ROWS8)])

    @pl.when(sid == 15)
    def _():
        pltpu.sync_copy(acc.at[pl.ds(16 * ROWS8, N - 16 * ROWS8)],
                        out_hbm.at[cid, pl.ds(16 * ROWS8, N - 16 * ROWS8)])


def _scatter(msg, dst, zeros_nd, e0):
    ne = msg.shape[0]
    mesh = plsc.VectorSubcoreMesh(core_axis_name="c", subcore_axis_name="s")
    return pl.kernel(
        functools.partial(_scatter_body, e0 // C, ne // C),
        out_type=(
            jax.ShapeDtypeStruct((2, N, D), jnp.float32),
            jax.ShapeDtypeStruct((NWORK * N,), jnp.float32),
        ),
        mesh=mesh,
        scratch_types=[
            pltpu.VMEM((C,), jnp.int32),
            pltpu.VMEM((C, D), jnp.float32),
            pltpu.VMEM((N,), jnp.float32),
            pltpu.VMEM_SHARED((N, D), jnp.float32),
        ],
        compiler_params=pltpu.CompilerParams(needs_layout_passes=False),
    )(msg, dst, zeros_nd)


_R5 = 1000


def _final_body(acca_ref, accb_ref, hista_ref, histb_ref, h_ref, out_ref):
    sums = acca_ref[0] + acca_ref[1] + accb_ref[0] + accb_ref[1]
    cnt = (jnp.sum(hista_ref[0], axis=0)
           + jnp.sum(histb_ref[0], axis=0))[:, None]
    agg = sums / jnp.maximum(cnt, 1.0)
    out_ref[...] = SKIP_W * agg + (1.0 - SKIP_W) * h_ref[...]


def _final(acc_a, acc_b, hists_a, hists_b, lig_h):
    ha = hists_a.reshape(NWORK, N // _R5, _R5).transpose(1, 0, 2)
    hb = hists_b.reshape(NWORK, N // _R5, _R5).transpose(1, 0, 2)
    return pl.pallas_call(
        _final_body,
        grid=(N // _R5,),
        in_specs=[
            pl.BlockSpec((2, _R5, D), lambda i: (0, i, 0)),
            pl.BlockSpec((2, _R5, D), lambda i: (0, i, 0)),
            pl.BlockSpec((1, NWORK, _R5), lambda i: (i, 0, 0)),
            pl.BlockSpec((1, NWORK, _R5), lambda i: (i, 0, 0)),
            pl.BlockSpec((_R5, D), lambda i: (i, 0)),
        ],
        out_specs=pl.BlockSpec((_R5, D), lambda i: (i, 0)),
        out_shape=jax.ShapeDtypeStruct((N, D), jnp.float32),
    )(acc_a, acc_b, ha, hb, lig_h)


def kernel(lig_h, lig_coords, lig_edge_index, lig_edge_attr, W1, b1,
           ln_gamma, ln_beta, W2, b2):
    src = lig_edge_index[0]
    dst = lig_edge_index[1]

    wa = W1[0:D]
    wb = W1[D:2 * D]
    w1cd = jnp.concatenate([W1[2 * D:2 * D + DE], W1[2 * D + DE:],
                            b1[None, :]], axis=0)
    coef = jnp.array([[-1.0 / s for s in SIGMAS] + [0.0]], jnp.float32)
    coords4 = jnp.pad(lig_coords, ((0, 0), (0, 1))).reshape(-1)

    d2 = _dist(src, dst, coords4)
    d2x16 = jnp.broadcast_to(d2[:, None], (E, DE))

    pa, pb = _node_pre(lig_h, wa, wb)
    zeros_nd = jnp.zeros((N, D), jnp.float32)

    EH = E // 2
    gam, bet, b2r = ln_gamma[None, :], ln_beta[None, :], b2[None, :]

    ex1 = _extra(d2x16, lig_edge_attr, coef, w1cd, 0, EH)
    ex2 = _extra(d2x16, lig_edge_attr, coef, w1cd, EH, EH)
    gs1, gd1 = _gather(pa, pb, src, dst, 0, EH)
    gs2, gd2 = _gather(pa, pb, src, dst, EH, EH)

    msg1 = _edge_mlp(gs1, gd1, ex1, W2, gam, bet, b2r)
    acc1, hists1 = _scatter(msg1, dst, zeros_nd, 0)
    msg2 = _edge_mlp(gs2, gd2, ex2, W2, gam, bet, b2r)
    acc2, hists2 = _scatter(msg2, dst, zeros_nd, EH)

    return _final(acc1, acc2, hists1, hists2, lig_h)


# speedup vs baseline: 1.2075x; 1.0160x over previous
import functools

import jax
import jax.numpy as jnp
from jax import lax
from jax.experimental import pallas as pl
from jax.experimental.pallas import tpu as pltpu
from jax.experimental.pallas import tpu_sc as plsc

N = 10000
E = 320000
D = 128
DE = 16
SIGMAS = [1.5 ** x for x in range(15)]
NEG_SLOPE = 0.01
SKIP_W = 0.5

C = 128
NCHUNKS = E // C
NWORK = 32
STEPS = (NCHUNKS + NWORK - 1) // NWORK
ROWS8 = 624

_R1 = 1000


def _node_pre_body(h_ref, wa_ref, wb_ref, oa_ref, ob_ref):
    h = h_ref[...]
    oa_ref[...] = jnp.dot(h, wa_ref[...], preferred_element_type=jnp.float32)
    ob_ref[...] = jnp.dot(h, wb_ref[...], preferred_element_type=jnp.float32)


def _node_pre(lig_h, wa, wb):
    return pl.pallas_call(
        _node_pre_body,
        grid=(N // _R1,),
        in_specs=[
            pl.BlockSpec((_R1, D), lambda i: (i, 0)),
            pl.BlockSpec((D, D), lambda i: (0, 0)),
            pl.BlockSpec((D, D), lambda i: (0, 0)),
        ],
        out_specs=[
            pl.BlockSpec((_R1, D), lambda i: (i, 0)),
            pl.BlockSpec((_R1, D), lambda i: (i, 0)),
        ],
        out_shape=[
            jax.ShapeDtypeStruct((N, D), jnp.float32),
            jax.ShapeDtypeStruct((N, D), jnp.float32),
        ],
    )(lig_h, wa, wb)


_CB = 1000
_DSTEPS = E // (NWORK * _CB)


def _dist_body(src_hbm, dst_hbm, c4_hbm, d2_hbm, idxs, idxd, cflat, d2buf):
    cid = lax.axis_index("c")
    sid = lax.axis_index("s")
    wid = sid * 2 + cid

    pltpu.sync_copy(c4_hbm, cflat)

    def step(k, carry):
        base = (wid + k * NWORK) * _CB
        pltpu.sync_copy(src_hbm.at[pl.ds(base, _CB)], idxs)
        pltpu.sync_copy(dst_hbm.at[pl.ds(base, _CB)], idxd)

        def group(g, carry2):
            o = g * 16
            s16 = idxs[pl.ds(o, 16)] * 4
            t16 = idxd[pl.ds(o, 16)] * 4
            dx = (plsc.load_gather(cflat, [s16])
                  - plsc.load_gather(cflat, [t16]))
            dy = (plsc.load_gather(cflat, [s16 + 1])
                  - plsc.load_gather(cflat, [t16 + 1]))
            dz = (plsc.load_gather(cflat, [s16 + 2])
                  - plsc.load_gather(cflat, [t16 + 2]))
            d2buf[pl.ds(o, 16)] = dx * dx + dy * dy + dz * dz
            return carry2

        lax.fori_loop(0, _CB // 16, group, 0, unroll=5)
        pltpu.sync_copy(d2buf, d2_hbm.at[pl.ds(base, _CB)])
        return carry

    lax.fori_loop(0, _DSTEPS, step, 0)


def _dist(src, dst, coords4):
    mesh = plsc.VectorSubcoreMesh(core_axis_name="c", subcore_axis_name="s")
    return pl.kernel(
        _dist_body,
        out_type=jax.ShapeDtypeStruct((E,), jnp.float32),
        mesh=mesh,
        scratch_types=[
            pltpu.VMEM((_CB,), jnp.int32),
            pltpu.VMEM((_CB,), jnp.int32),
            pltpu.VMEM((4 * N,), jnp.float32),
            pltpu.VMEM((_CB,), jnp.float32),
        ],
        compiler_params=pltpu.CompilerParams(needs_layout_passes=False),
    )(src, dst, coords4)


def _gather_body(chunk0, nchunks, ta_hbm, tb_hbm, src_hbm, dst_hbm,
                 gs_hbm, gd_hbm, idxs, idxd, bufs, bufd, sems, semd):
    cid = lax.axis_index("c")
    sid = lax.axis_index("s")
    wid = sid * 2 + cid

    def step(k, carry):
        chunk = wid + k * NWORK

        @pl.when(chunk < nchunks)
        def _():
            gbase = (chunk0 + chunk) * C
            base = chunk * C
            pltpu.sync_copy(src_hbm.at[pl.ds(gbase, C)], idxs)
            pltpu.sync_copy(dst_hbm.at[pl.ds(gbase, C)], idxd)
            cp1 = pltpu.async_copy(ta_hbm.at[idxs], bufs, sems)
            cp2 = pltpu.async_copy(tb_hbm.at[idxd], bufd, semd)
            cp1.wait()
            cp2.wait()
            pltpu.sync_copy(bufs, gs_hbm.at[pl.ds(base, C)])
            pltpu.sync_copy(bufd, gd_hbm.at[pl.ds(base, C)])

        return carry

    lax.fori_loop(0, (nchunks + NWORK - 1) // NWORK, step, 0)


def _gather(ta, tb, src, dst, e0, ne):
    mesh = plsc.VectorSubcoreMesh(core_axis_name="c", subcore_axis_name="s")
    return pl.kernel(
        functools.partial(_gather_body, e0 // C, ne // C),
        out_type=(
            jax.ShapeDtypeStruct((ne, D), jnp.float32),
            jax.ShapeDtypeStruct((ne, D), jnp.float32),
        ),
        mesh=mesh,
        scratch_types=[
            pltpu.VMEM((C,), jnp.int32),
            pltpu.VMEM((C,), jnp.int32),
            pltpu.VMEM((C, D), jnp.float32),
            pltpu.VMEM((C, D), jnp.float32),
            pltpu.SemaphoreType.DMA,
            pltpu.SemaphoreType.DMA,
        ],
        compiler_params=pltpu.CompilerParams(needs_layout_passes=False),
    )(ta, tb, src, dst)


_B3 = 3200


def _extra_body(d2_ref, ea_ref, coef_ref, w1cd_ref, out_ref):
    rbf1 = jnp.exp(d2_ref[...].astype(jnp.float32) * coef_ref[...])
    cfeat = jnp.concatenate([ea_ref[...], rbf1], axis=1)
    out_ref[...] = jnp.dot(cfeat, w1cd_ref[...],
                           preferred_element_type=jnp.float32).astype(jnp.bfloat16)


def _extra(d2x16, eattr, coef, w1cd, e0, ne):
    blk0 = e0 // _B3
    return pl.pallas_call(
        _extra_body,
        grid=(ne // _B3,),
        in_specs=[
            pl.BlockSpec((_B3, DE), lambda i: (blk0 + i, 0)),
            pl.BlockSpec((_B3, DE), lambda i: (blk0 + i, 0)),
            pl.BlockSpec((1, DE), lambda i: (0, 0)),
            pl.BlockSpec((32, D), lambda i: (0, 0)),
        ],
        out_specs=pl.BlockSpec((_B3, D), lambda i: (i, 0)),
        out_shape=jax.ShapeDtypeStruct((ne, D), jnp.bfloat16),
    )(d2x16, eattr, coef, w1cd)


def _edge_mlp_body(gs_ref, gd_ref, ex_ref, w2_ref, gam_ref, bet_ref,
                   b2_ref, out_ref):
    pre = gs_ref[...] + gd_ref[...] + ex_ref[...].astype(jnp.float32)
    pre = jnp.where(pre >= 0, pre, NEG_SLOPE * pre)
    mu = jnp.mean(pre, axis=1, keepdims=True)
    xc = pre - mu
    var = jnp.mean(xc * xc, axis=1, keepdims=True)
    hdd = xc * lax.rsqrt(var + 1e-5) * gam_ref[...] + bet_ref[...]
    msg = jnp.dot(hdd, w2_ref[...], preferred_element_type=jnp.float32)
    out_ref[...] = msg + b2_ref[...]


def _edge_mlp(gs, gd, extra, w2, gamma, beta, b2):
    ne = gs.shape[0]
    return pl.pallas_call(
        _edge_mlp_body,
        grid=(ne // _B3,),
        in_specs=[
            pl.BlockSpec((_B3, D), lambda i: (i, 0)),
            pl.BlockSpec((_B3, D), lambda i: (i, 0)),
            pl.BlockSpec((_B3, D), lambda i: (i, 0)),
            pl.BlockSpec((D, D), lambda i: (0, 0)),
            pl.BlockSpec((1, D), lambda i: (0, 0)),
            pl.BlockSpec((1, D), lambda i: (0, 0)),
            pl.BlockSpec((1, D), lambda i: (0, 0)),
        ],
        out_specs=pl.BlockSpec((_B3, D), lambda i: (i, 0)),
        out_shape=jax.ShapeDtypeStruct((ne, D), jnp.float32),
    )(gs, gd, extra, w2, gamma, beta, b2)


def _scatter_body(chunk0, nchunks, msg_hbm, dst_hbm, zer_hbm, out_hbm,
                  hist_hbm, idxv, bufv, hist, acc):
    cid = lax.axis_index("c")
    sid = lax.axis_index("s")
    wid = sid * 2 + cid
    row0 = sid * ROWS8

    pltpu.sync_copy(zer_hbm.at[pl.ds(row0, ROWS8)],
                    acc.at[pl.ds(row0, ROWS8)])

    @pl.when(sid == 15)
    def _():
        pltpu.sync_copy(zer_hbm.at[pl.ds(16 * ROWS8, N - 16 * ROWS8)],
                        acc.at[pl.ds(16 * ROWS8, N - 16 * ROWS8)])

    def zstep(i, carry):
        hist[pl.ds(i * 16, 16)] = jnp.zeros((16,), jnp.float32)
        return carry

    lax.fori_loop(0, N // 16, zstep, 0)
    plsc.subcore_barrier()

    ones16 = jnp.ones((16,), jnp.float32)

    def step(k, carry):
        chunk = wid + k * NWORK

        @pl.when(chunk < nchunks)
        def _():
            base = chunk * C
            pltpu.sync_copy(dst_hbm.at[pl.ds((chunk0 + chunk) * C, C)], idxv)
            pltpu.sync_copy(msg_hbm.at[pl.ds(base, C)], bufv)
            pltpu.sync_copy(bufv, acc.at[idxv], add=True)
            for g in range(C // 16):
                i16 = idxv[pl.ds(g * 16, 16)]
                plsc.addupdate_scatter(hist, [i16], ones16)

        return carry

    lax.fori_loop(0, (nchunks + NWORK - 1) // NWORK, step, 0)
    pltpu.sync_copy(hist, hist_hbm.at[pl.ds(wid * N, N)])
    plsc.subcore_barrier()
    pltpu.sync_copy(acc.at[pl.ds(row0, ROWS8)],
                    out_hbm.at[cid, pl.ds(row0, ROWS8)])

    @pl.when(sid == 15)
    def _():
        pltpu.sync_copy(acc.at[pl.ds(16 * ROWS8, N - 16 * ROWS8)],
                        out_hbm.at[cid, pl.ds(16 * ROWS8, N - 16 * ROWS8)])


def _scatter(msg, dst, zeros_nd, e0):
    ne = msg.shape[0]
    mesh = plsc.VectorSubcoreMesh(core_axis_name="c", subcore_axis_name="s")
    return pl.kernel(
        functools.partial(_scatter_body, e0 // C, ne // C),
        out_type=(
            jax.ShapeDtypeStruct((2, N, D), jnp.float32),
            jax.ShapeDtypeStruct((NWORK * N,), jnp.float32),
        ),
        mesh=mesh,
        scratch_types=[
            pltpu.VMEM((C,), jnp.int32),
            pltpu.VMEM((C, D), jnp.float32),
            pltpu.VMEM((N,), jnp.float32),
            pltpu.VMEM_SHARED((N, D), jnp.float32),
        ],
        compiler_params=pltpu.CompilerParams(needs_layout_passes=False),
    )(msg, dst, zeros_nd)


_R5 = 1000


def _final_body(acca_ref, accb_ref, hista_ref, histb_ref, h_ref, out_ref):
    sums = acca_ref[0] + acca_ref[1] + accb_ref[0] + accb_ref[1]
    cnt = (jnp.sum(hista_ref[0], axis=0)
           + jnp.sum(histb_ref[0], axis=0))[:, None]
    agg = sums / jnp.maximum(cnt, 1.0)
    out_ref[...] = SKIP_W * agg + (1.0 - SKIP_W) * h_ref[...]


def _final(acc_a, acc_b, hists_a, hists_b, lig_h):
    ha = hists_a.reshape(NWORK, N // _R5, _R5).transpose(1, 0, 2)
    hb = hists_b.reshape(NWORK, N // _R5, _R5).transpose(1, 0, 2)
    return pl.pallas_call(
        _final_body,
        grid=(N // _R5,),
        in_specs=[
            pl.BlockSpec((2, _R5, D), lambda i: (0, i, 0)),
            pl.BlockSpec((2, _R5, D), lambda i: (0, i, 0)),
            pl.BlockSpec((1, NWORK, _R5), lambda i: (i, 0, 0)),
            pl.BlockSpec((1, NWORK, _R5), lambda i: (i, 0, 0)),
            pl.BlockSpec((_R5, D), lambda i: (i, 0)),
        ],
        out_specs=pl.BlockSpec((_R5, D), lambda i: (i, 0)),
        out_shape=jax.ShapeDtypeStruct((N, D), jnp.float32),
    )(acc_a, acc_b, ha, hb, lig_h)


def kernel(lig_h, lig_coords, lig_edge_index, lig_edge_attr, W1, b1,
           ln_gamma, ln_beta, W2, b2):
    src = lig_edge_index[0]
    dst = lig_edge_index[1]

    wa = W1[0:D]
    wb = W1[D:2 * D]
    w1cd = jnp.concatenate([W1[2 * D:2 * D + DE], W1[2 * D + DE:],
                            b1[None, :]], axis=0)
    coef = jnp.array([[-1.0 / s for s in SIGMAS] + [0.0]], jnp.float32)
    coords4 = jnp.pad(lig_coords, ((0, 0), (0, 1))).reshape(-1)

    d2 = _dist(src, dst, coords4)
    d2x16 = jnp.broadcast_to(d2.astype(jnp.bfloat16)[:, None], (E, DE))

    pa, pb = _node_pre(lig_h, wa, wb)
    zeros_nd = jnp.zeros((N, D), jnp.float32)

    EH = E // 2
    gam, bet, b2r = ln_gamma[None, :], ln_beta[None, :], b2[None, :]

    ex1 = _extra(d2x16, lig_edge_attr, coef, w1cd, 0, EH)
    ex2 = _extra(d2x16, lig_edge_attr, coef, w1cd, EH, EH)
    gs1, gd1 = _gather(pa, pb, src, dst, 0, EH)
    gs2, gd2 = _gather(pa, pb, src, dst, EH, EH)

    msg1 = _edge_mlp(gs1, gd1, ex1, W2, gam, bet, b2r)
    acc1, hists1 = _scatter(msg1, dst, zeros_nd, 0)
    msg2 = _edge_mlp(gs2, gd2, ex2, W2, gam, bet, b2r)
    acc2, hists2 = _scatter(msg2, dst, zeros_nd, EH)

    return _final(acc1, acc2, hists1, hists2, lig_h)


# speedup vs baseline: 1.2236x; 1.0134x over previous
import functools

import jax
import jax.numpy as jnp
from jax import lax
from jax.experimental import pallas as pl
from jax.experimental.pallas import tpu as pltpu
from jax.experimental.pallas import tpu_sc as plsc

N = 10000
E = 320000
D = 128
DE = 16
SIGMAS = [1.5 ** x for x in range(15)]
NEG_SLOPE = 0.01
SKIP_W = 0.5

C = 128
NCHUNKS = E // C
NWORK = 32
STEPS = (NCHUNKS + NWORK - 1) // NWORK
ROWS8 = 624

_R1 = 1000


def _node_pre_body(h_ref, wa_ref, wb_ref, oa_ref, ob_ref):
    h = h_ref[...]
    oa_ref[...] = jnp.dot(h, wa_ref[...], preferred_element_type=jnp.float32)
    ob_ref[...] = jnp.dot(h, wb_ref[...], preferred_element_type=jnp.float32)


def _node_pre(lig_h, wa, wb):
    return pl.pallas_call(
        _node_pre_body,
        grid=(N // _R1,),
        in_specs=[
            pl.BlockSpec((_R1, D), lambda i: (i, 0)),
            pl.BlockSpec((D, D), lambda i: (0, 0)),
            pl.BlockSpec((D, D), lambda i: (0, 0)),
        ],
        out_specs=[
            pl.BlockSpec((_R1, D), lambda i: (i, 0)),
            pl.BlockSpec((_R1, D), lambda i: (i, 0)),
        ],
        out_shape=[
            jax.ShapeDtypeStruct((N, D), jnp.float32),
            jax.ShapeDtypeStruct((N, D), jnp.float32),
        ],
    )(lig_h, wa, wb)


_CB = 1000
_DSTEPS = E // (NWORK * _CB)


def _dist_body(src_hbm, dst_hbm, c4_hbm, d2_hbm, idxs, idxd, cflat, d2buf):
    cid = lax.axis_index("c")
    sid = lax.axis_index("s")
    wid = sid * 2 + cid

    pltpu.sync_copy(c4_hbm, cflat)

    def step(k, carry):
        base = (wid + k * NWORK) * _CB
        pltpu.sync_copy(src_hbm.at[pl.ds(base, _CB)], idxs)
        pltpu.sync_copy(dst_hbm.at[pl.ds(base, _CB)], idxd)

        def group(g, carry2):
            o = g * 16
            s16 = idxs[pl.ds(o, 16)] * 4
            t16 = idxd[pl.ds(o, 16)] * 4
            dx = (plsc.load_gather(cflat, [s16])
                  - plsc.load_gather(cflat, [t16]))
            dy = (plsc.load_gather(cflat, [s16 + 1])
                  - plsc.load_gather(cflat, [t16 + 1]))
            dz = (plsc.load_gather(cflat, [s16 + 2])
                  - plsc.load_gather(cflat, [t16 + 2]))
            d2buf[pl.ds(o, 16)] = dx * dx + dy * dy + dz * dz
            return carry2

        lax.fori_loop(0, _CB // 16, group, 0, unroll=5)
        pltpu.sync_copy(d2buf, d2_hbm.at[pl.ds(base, _CB)])
        return carry

    lax.fori_loop(0, _DSTEPS, step, 0)


def _dist(src, dst, coords4):
    mesh = plsc.VectorSubcoreMesh(core_axis_name="c", subcore_axis_name="s")
    return pl.kernel(
        _dist_body,
        out_type=jax.ShapeDtypeStruct((E,), jnp.float32),
        mesh=mesh,
        scratch_types=[
            pltpu.VMEM((_CB,), jnp.int32),
            pltpu.VMEM((_CB,), jnp.int32),
            pltpu.VMEM((4 * N,), jnp.float32),
            pltpu.VMEM((_CB,), jnp.float32),
        ],
        compiler_params=pltpu.CompilerParams(needs_layout_passes=False),
    )(src, dst, coords4)


def _gather_body(chunk0, nchunks, ta_hbm, tb_hbm, src_hbm, dst_hbm,
                 gs_hbm, gd_hbm,
                 idx0s, idx0d, idx1s, idx1d, buf0s, buf0d, buf1s, buf1d,
                 sems, semd, ws0, wd0, ws1, wd1):
    cid = lax.axis_index("c")
    sid = lax.axis_index("s")
    wid = sid * 2 + cid
    idxs_ = (idx0s, idx1s)
    idxd_ = (idx0d, idx1d)
    bufs_ = (buf0s, buf1s)
    bufd_ = (buf0d, buf1d)
    ws_ = (ws0, ws1)
    wd_ = (wd0, wd1)

    def step(k2, carry):
        for j in (0, 1):
            k = k2 * 2 + j
            chunk = wid + k * NWORK

            @pl.when(chunk < nchunks)
            def _(j=j, k=k, chunk=chunk):
                @pl.when(k >= 2)
                def _():
                    pltpu.make_async_copy(
                        bufs_[j], gs_hbm.at[pl.ds(0, C)], ws_[j]).wait()
                    pltpu.make_async_copy(
                        bufd_[j], gd_hbm.at[pl.ds(0, C)], wd_[j]).wait()

                gbase = (chunk0 + chunk) * C
                base = chunk * C
                pltpu.sync_copy(src_hbm.at[pl.ds(gbase, C)], idxs_[j])
                pltpu.sync_copy(dst_hbm.at[pl.ds(gbase, C)], idxd_[j])
                cp1 = pltpu.async_copy(ta_hbm.at[idxs_[j]], bufs_[j], sems)
                cp2 = pltpu.async_copy(tb_hbm.at[idxd_[j]], bufd_[j], semd)
                cp1.wait()
                cp2.wait()
                pltpu.async_copy(bufs_[j], gs_hbm.at[pl.ds(base, C)], ws_[j])
                pltpu.async_copy(bufd_[j], gd_hbm.at[pl.ds(base, C)], wd_[j])

        return carry

    steps2 = ((nchunks + NWORK - 1) // NWORK + 1) // 2
    lax.fori_loop(0, steps2, step, 0)
    for j in (0, 1):
        pltpu.make_async_copy(bufs_[j], gs_hbm.at[pl.ds(0, C)], ws_[j]).wait()
        pltpu.make_async_copy(bufd_[j], gd_hbm.at[pl.ds(0, C)], wd_[j]).wait()


def _gather(ta, tb, src, dst, e0, ne):
    mesh = plsc.VectorSubcoreMesh(core_axis_name="c", subcore_axis_name="s")
    return pl.kernel(
        functools.partial(_gather_body, e0 // C, ne // C),
        out_type=(
            jax.ShapeDtypeStruct((ne, D), jnp.float32),
            jax.ShapeDtypeStruct((ne, D), jnp.float32),
        ),
        mesh=mesh,
        scratch_types=[
            pltpu.VMEM((C,), jnp.int32),
            pltpu.VMEM((C,), jnp.int32),
            pltpu.VMEM((C,), jnp.int32),
            pltpu.VMEM((C,), jnp.int32),
            pltpu.VMEM((C, D), jnp.float32),
            pltpu.VMEM((C, D), jnp.float32),
            pltpu.VMEM((C, D), jnp.float32),
            pltpu.VMEM((C, D), jnp.float32),
            pltpu.SemaphoreType.DMA,
            pltpu.SemaphoreType.DMA,
            pltpu.SemaphoreType.DMA,
            pltpu.SemaphoreType.DMA,
            pltpu.SemaphoreType.DMA,
            pltpu.SemaphoreType.DMA,
        ],
        compiler_params=pltpu.CompilerParams(needs_layout_passes=False),
    )(ta, tb, src, dst)


_B3 = 3200


def _extra_body(d2_ref, ea_ref, coef_ref, w1cd_ref, out_ref):
    rbf1 = jnp.exp(d2_ref[...].astype(jnp.float32) * coef_ref[...])
    cfeat = jnp.concatenate([ea_ref[...], rbf1], axis=1)
    out_ref[...] = jnp.dot(cfeat, w1cd_ref[...],
                           preferred_element_type=jnp.float32).astype(jnp.bfloat16)


def _extra(d2x16, eattr, coef, w1cd, e0, ne):
    blk0 = e0 // _B3
    return pl.pallas_call(
        _extra_body,
        grid=(ne // _B3,),
        in_specs=[
            pl.BlockSpec((_B3, DE), lambda i: (blk0 + i, 0)),
            pl.BlockSpec((_B3, DE), lambda i: (blk0 + i, 0)),
            pl.BlockSpec((1, DE), lambda i: (0, 0)),
            pl.BlockSpec((32, D), lambda i: (0, 0)),
        ],
        out_specs=pl.BlockSpec((_B3, D), lambda i: (i, 0)),
        out_shape=jax.ShapeDtypeStruct((ne, D), jnp.bfloat16),
    )(d2x16, eattr, coef, w1cd)


def _edge_mlp_body(gs_ref, gd_ref, ex_ref, w2_ref, gam_ref, bet_ref,
                   b2_ref, out_ref):
    pre = gs_ref[...] + gd_ref[...] + ex_ref[...].astype(jnp.float32)
    pre = jnp.where(pre >= 0, pre, NEG_SLOPE * pre)
    mu = jnp.mean(pre, axis=1, keepdims=True)
    xc = pre - mu
    var = jnp.mean(xc * xc, axis=1, keepdims=True)
    hdd = xc * lax.rsqrt(var + 1e-5) * gam_ref[...] + bet_ref[...]
    msg = jnp.dot(hdd, w2_ref[...], preferred_element_type=jnp.float32)
    out_ref[...] = msg + b2_ref[...]


def _edge_mlp(gs, gd, extra, w2, gamma, beta, b2):
    ne = gs.shape[0]
    return pl.pallas_call(
        _edge_mlp_body,
        grid=(ne // _B3,),
        in_specs=[
            pl.BlockSpec((_B3, D), lambda i: (i, 0)),
            pl.BlockSpec((_B3, D), lambda i: (i, 0)),
            pl.BlockSpec((_B3, D), lambda i: (i, 0)),
            pl.BlockSpec((D, D), lambda i: (0, 0)),
            pl.BlockSpec((1, D), lambda i: (0, 0)),
            pl.BlockSpec((1, D), lambda i: (0, 0)),
            pl.BlockSpec((1, D), lambda i: (0, 0)),
        ],
        out_specs=pl.BlockSpec((_B3, D), lambda i: (i, 0)),
        out_shape=jax.ShapeDtypeStruct((ne, D), jnp.float32),
    )(gs, gd, extra, w2, gamma, beta, b2)


def _scatter_body(chunk0, nchunks, msg_hbm, dst_hbm, zer_hbm, out_hbm,
                  hist_hbm, idxv, bufv, hist, acc):
    cid = lax.axis_index("c")
    sid = lax.axis_index("s")
    wid = sid * 2 + cid
    row0 = sid * ROWS8

    pltpu.sync_copy(zer_hbm.at[pl.ds(row0, ROWS8)],
                    acc.at[pl.ds(row0, ROWS8)])

    @pl.when(sid == 15)
    def _():
        pltpu.sync_copy(zer_hbm.at[pl.ds(16 * ROWS8, N - 16 * ROWS8)],
                        acc.at[pl.ds(16 * ROWS8, N - 16 * ROWS8)])

    def zstep(i, carry):
        hist[pl.ds(i * 16, 16)] = jnp.zeros((16,), jnp.float32)
        return carry

    lax.fori_loop(0, N // 16, zstep, 0)
    plsc.subcore_barrier()

    ones16 = jnp.ones((16,), jnp.float32)

    def step(k, carry):
        chunk = wid + k * NWORK

        @pl.when(chunk < nchunks)
        def _():
            base = chunk * C
            pltpu.sync_copy(dst_hbm.at[pl.ds((chunk0 + chunk) * C, C)], idxv)
            pltpu.sync_copy(msg_hbm.at[pl.ds(base, C)], bufv)
            pltpu.sync_copy(bufv, acc.at[idxv], add=True)
            for g in range(C // 16):
                i16 = idxv[pl.ds(g * 16, 16)]
                plsc.addupdate_scatter(hist, [i16], ones16)

        return carry

    lax.fori_loop(0, (nchunks + NWORK - 1) // NWORK, step, 0)
    pltpu.sync_copy(hist, hist_hbm.at[pl.ds(wid * N, N)])
    plsc.subcore_barrier()
    pltpu.sync_copy(acc.at[pl.ds(row0, ROWS8)],
                    out_hbm.at[cid, pl.ds(row0, ROWS8)])

    @pl.when(sid == 15)
    def _():
        pltpu.sync_copy(acc.at[pl.ds(16 * ROWS8, N - 16 * ROWS8)],
                        out_hbm.at[cid, pl.ds(16 * ROWS8, N - 16 * ROWS8)])


def _scatter(msg, dst, zeros_nd, e0):
    ne = msg.shape[0]
    mesh = plsc.VectorSubcoreMesh(core_axis_name="c", subcore_axis_name="s")
    return pl.kernel(
        functools.partial(_scatter_body, e0 // C, ne // C),
        out_type=(
            jax.ShapeDtypeStruct((2, N, D), jnp.float32),
            jax.ShapeDtypeStruct((NWORK * N,), jnp.float32),
        ),
        mesh=mesh,
        scratch_types=[
            pltpu.VMEM((C,), jnp.int32),
            pltpu.VMEM((C, D), jnp.float32),
            pltpu.VMEM((N,), jnp.float32),
            pltpu.VMEM_SHARED((N, D), jnp.float32),
        ],
        compiler_params=pltpu.CompilerParams(needs_layout_passes=False),
    )(msg, dst, zeros_nd)


_R5 = 1000


def _final_body(acca_ref, accb_ref, hista_ref, histb_ref, h_ref, out_ref):
    sums = acca_ref[0] + acca_ref[1] + accb_ref[0] + accb_ref[1]
    cnt = (jnp.sum(hista_ref[0], axis=0)
           + jnp.sum(histb_ref[0], axis=0))[:, None]
    agg = sums / jnp.maximum(cnt, 1.0)
    out_ref[...] = SKIP_W * agg + (1.0 - SKIP_W) * h_ref[...]


def _final(acc_a, acc_b, hists_a, hists_b, lig_h):
    ha = hists_a.reshape(NWORK, N // _R5, _R5).transpose(1, 0, 2)
    hb = hists_b.reshape(NWORK, N // _R5, _R5).transpose(1, 0, 2)
    return pl.pallas_call(
        _final_body,
        grid=(N // _R5,),
        in_specs=[
            pl.BlockSpec((2, _R5, D), lambda i: (0, i, 0)),
            pl.BlockSpec((2, _R5, D), lambda i: (0, i, 0)),
            pl.BlockSpec((1, NWORK, _R5), lambda i: (i, 0, 0)),
            pl.BlockSpec((1, NWORK, _R5), lambda i: (i, 0, 0)),
            pl.BlockSpec((_R5, D), lambda i: (i, 0)),
        ],
        out_specs=pl.BlockSpec((_R5, D), lambda i: (i, 0)),
        out_shape=jax.ShapeDtypeStruct((N, D), jnp.float32),
    )(acc_a, acc_b, ha, hb, lig_h)


def kernel(lig_h, lig_coords, lig_edge_index, lig_edge_attr, W1, b1,
           ln_gamma, ln_beta, W2, b2):
    src = lig_edge_index[0]
    dst = lig_edge_index[1]

    wa = W1[0:D]
    wb = W1[D:2 * D]
    w1cd = jnp.concatenate([W1[2 * D:2 * D + DE], W1[2 * D + DE:],
                            b1[None, :]], axis=0)
    coef = jnp.array([[-1.0 / s for s in SIGMAS] + [0.0]], jnp.float32)
    coords4 = jnp.pad(lig_coords, ((0, 0), (0, 1))).reshape(-1)

    d2 = _dist(src, dst, coords4)
    d2x16 = jnp.broadcast_to(d2.astype(jnp.bfloat16)[:, None], (E, DE))

    pa, pb = _node_pre(lig_h, wa, wb)
    zeros_nd = jnp.zeros((N, D), jnp.float32)

    EH = E // 2
    gam, bet, b2r = ln_gamma[None, :], ln_beta[None, :], b2[None, :]

    ex1 = _extra(d2x16, lig_edge_attr, coef, w1cd, 0, EH)
    ex2 = _extra(d2x16, lig_edge_attr, coef, w1cd, EH, EH)
    gs1, gd1 = _gather(pa, pb, src, dst, 0, EH)
    gs2, gd2 = _gather(pa, pb, src, dst, EH, EH)

    msg1 = _edge_mlp(gs1, gd1, ex1, W2, gam, bet, b2r)
    acc1, hists1 = _scatter(msg1, dst, zeros_nd, 0)
    msg2 = _edge_mlp(gs2, gd2, ex2, W2, gam, bet, b2r)
    acc2, hists2 = _scatter(msg2, dst, zeros_nd, EH)

    return _final(acc1, acc2, hists1, hists2, lig_h)
